# baseline (speedup 1.0000x reference)
import functools

import jax
import jax.numpy as jnp
from jax import lax
from jax.experimental import pallas as pl
from jax.experimental.pallas import tpu as pltpu
from jax.experimental.pallas import tpu_sc as plsc

ALPHA = 0.1
BETA = 0.5
NC = 2
NS = 16
LANES = 16
CH = 48
CHD = 128


def _sc_mesh():
    return plsc.VectorSubcoreMesh(
        core_axis_name="c", subcore_axis_name="s",
        num_cores=NC, num_subcores=NS)


_SC_PARAMS = pltpu.CompilerParams(needs_layout_passes=False)


def _make_deg_kernel(e_pad, n_pad):
    nw = NC * NS
    pt = e_pad // nw
    nit = pt // CHD
    slc = n_pad // NS

    @functools.partial(
        pl.kernel,
        out_type=jax.ShapeDtypeStruct((NC * n_pad,), jnp.float32),
        mesh=_sc_mesh(),
        compiler_params=_SC_PARAMS,
        scratch_types=[
            pltpu.VMEM((nit, CHD), jnp.int32),
            pltpu.VMEM((nit, CHD), jnp.float32),
            pltpu.VMEM((slc,), jnp.float32),
            pltpu.VMEM_SHARED((n_pad,), jnp.float32),
        ],
    )
    def deg_kernel(col_hbm, w_hbm, out_hbm, col2_v, w2_v, zb, s_deg):
        c = lax.axis_index("c")
        s = lax.axis_index("s")
        wid = c * NS + s

        def zero_body(i, carry):
            zb[pl.ds(i * LANES, LANES)] = jnp.zeros((LANES,), jnp.float32)
            return carry
        lax.fori_loop(0, slc // LANES, zero_body, 0)
        pltpu.sync_copy(zb, s_deg.at[pl.ds(s * slc, slc)])
        pltpu.sync_copy(col_hbm.at[wid], col2_v)
        pltpu.sync_copy(w_hbm.at[wid], w2_v)
        plsc.subcore_barrier()

        def edge_body(it, carry):
            pltpu.sync_copy(w2_v.at[it], s_deg.at[col2_v.at[it]], add=True)
            return carry
        lax.fori_loop(0, nit, edge_body, 0)
        plsc.subcore_barrier()
        pltpu.sync_copy(s_deg.at[pl.ds(s * slc, slc)],
                        out_hbm.at[pl.ds(c * n_pad + s * slc, slc)])

    return deg_kernel


def _make_norm_kernel(e_pad, n_pad):
    nw = NC * NS
    pt = e_pad // nw

    @functools.partial(
        pl.kernel,
        out_type=jax.ShapeDtypeStruct((nw, pt), jnp.float32),
        mesh=_sc_mesh(),
        compiler_params=_SC_PARAMS,
        scratch_types=[
            pltpu.VMEM((n_pad,), jnp.float32),
            pltpu.VMEM((pt,), jnp.int32),
            pltpu.VMEM((pt,), jnp.int32),
            pltpu.VMEM((pt,), jnp.float32),
            pltpu.VMEM((pt,), jnp.float32),
        ],
    )
    def norm_kernel(dinv_hbm, row_hbm, col_hbm, w_hbm, out_hbm,
                    dinv_v, row_v, col_v, w_v, nrm_v):
        c = lax.axis_index("c")
        s = lax.axis_index("s")
        wid = c * NS + s
        pltpu.sync_copy(dinv_hbm, dinv_v)
        pltpu.sync_copy(row_hbm.at[wid], row_v)
        pltpu.sync_copy(col_hbm.at[wid], col_v)
        pltpu.sync_copy(w_hbm.at[wid], w_v)

        def grp(g, carry):
            sl = pl.ds(g * LANES, LANES)
            dr = plsc.load_gather(dinv_v, [row_v[sl]])
            dc = plsc.load_gather(dinv_v, [col_v[sl]])
            nrm_v[sl] = dr * w_v[sl] * dc
            return carry
        lax.fori_loop(0, pt // LANES, grp, 0)
        pltpu.sync_copy(nrm_v, out_hbm.at[wid])

    return norm_kernel


def _make_layer_kernel(e_pad, n_pad, hid):
    nw = NC * NS
    pt = e_pad // nw
    nit = pt // CH
    assert nit % 8 == 0
    slc = n_pad // NS
    nz = slc // CH
    nz_rem = slc % CH
    kreg = hid // LANES

    @functools.partial(
        pl.kernel,
        out_type=jax.ShapeDtypeStruct((NC * n_pad, hid), jnp.float32),
        mesh=_sc_mesh(),
        compiler_params=_SC_PARAMS,
        scratch_types=(
            [pltpu.VMEM((CH,), jnp.int32) for _ in range(8)]
            + [pltpu.VMEM((CH,), jnp.int32) for _ in range(8)]
            + [pltpu.VMEM((CH,), jnp.float32) for _ in range(8)]
            + [pltpu.VMEM((CH, hid), jnp.float32) for _ in range(4)]
            + [pltpu.VMEM((CH, hid), jnp.float32) for _ in range(2)]
            + [pltpu.VMEM_SHARED((n_pad, hid), jnp.float32)]
            + [pltpu.SemaphoreType.DMA for _ in range(8)]
            + [pltpu.SemaphoreType.DMA for _ in range(4)]
            + [pltpu.SemaphoreType.DMA for _ in range(2)]
        ),
    )
    def layer_kernel(cur_hbm, row_hbm, col_hbm, nrm_hbm, out_hbm, *refs):
        rb = refs[0:8]
        cb = refs[8:16]
        nb_ = refs[16:24]
        gb = refs[24:28]
        sb = refs[28:30]
        s_agg = refs[30]
        esem = refs[31:39]
        gsem = refs[39:43]
        ssem = refs[43:45]
        c = lax.axis_index("c")
        s = lax.axis_index("s")
        wid = c * NS + s

        def fetch(t, q):
            return (pltpu.async_copy(row_hbm.at[wid * nit + t], rb[q],
                                     esem[q]),
                    pltpu.async_copy(col_hbm.at[wid * nit + t], cb[q],
                                     esem[q]),
                    pltpu.async_copy(nrm_hbm.at[wid * nit + t], nb_[q],
                                     esem[q]))

        for buf in sb:
            def zrow(i, carry, buf=buf):
                for k in range(kreg):
                    buf[i, pl.ds(k * LANES, LANES)] = jnp.zeros(
                        (LANES,), jnp.float32)
                return carry
            lax.fori_loop(0, CH, zrow, 0)
        for j in range(nz):
            pltpu.sync_copy(sb[0], s_agg.at[pl.ds(s * slc + j * CH, CH)])
        if nz_rem:
            pltpu.sync_copy(
                sb[0].at[pl.ds(0, nz_rem)],
                s_agg.at[pl.ds(s * slc + nz * CH, nz_rem)])
        plsc.subcore_barrier()

        for q in range(6):
            fetch(q, q)
        ewait = [(pltpu.make_async_copy(row_hbm.at[0], rb[q], esem[q]),
                  pltpu.make_async_copy(col_hbm.at[0], cb[q], esem[q]),
                  pltpu.make_async_copy(nrm_hbm.at[0], nb_[q], esem[q]))
                 for q in range(8)]
        for q in range(4):
            for p in ewait[q]:
                p.wait()
        cps = [pltpu.async_copy(cur_hbm.at[rb[q]], gb[q], gsem[q])
               for q in range(4)]
        dps = [pltpu.async_copy(sb[q], s_agg.at[cb[q]], ssem[q], add=True)
               for q in range(2)]

        def phase(j, carry):
            for b8 in range(8):
                it = 8 * j + b8
                q4 = b8 & 3
                q2 = b8 & 1
                cps[q4].wait()
                dps[q2].wait()

                @plsc.parallel_loop(0, CH, step=4, unroll=4)
                def scale(e0):
                    for du in range(4):
                        e = e0 + du
                        nrm = plsc.load_gather(
                            nb_[b8], [jnp.full((LANES,), e, jnp.int32)])
                        for k in range(kreg):
                            sl = pl.ds(k * LANES, LANES)
                            sb[q2][e, sl] = gb[q4][e, sl] * nrm

                pltpu.async_copy(sb[q2], s_agg.at[cb[b8]], ssem[q2],
                                 add=True)

                @pl.when(it + 4 < nit)
                def _():
                    for p in ewait[(b8 + 4) & 7]:
                        p.wait()
                    pltpu.async_copy(cur_hbm.at[rb[(b8 + 4) & 7]],
                                     gb[q4], gsem[q4])

                @pl.when(it + 6 < nit)
                def _():
                    fetch(it + 6, (b8 + 6) & 7)
            return carry
        lax.fori_loop(0, nit // 8, phase, 0)
        dps[0].wait()
        dps[1].wait()
        plsc.subcore_barrier()
        pltpu.sync_copy(s_agg.at[pl.ds(s * slc, slc)],
                        out_hbm.at[pl.ds(c * n_pad + s * slc, slc)])

    return layer_kernel


def _tc_dinv(deg2):
    n_pad = deg2.shape[1]

    def body(deg_ref, dinv_ref):
        deg = deg_ref[0:1, :] + deg_ref[1:2, :]
        dinv_ref[...] = jnp.where(
            deg > 0, lax.rsqrt(jnp.maximum(deg, 1e-12)), 0.0)

    return pl.pallas_call(
        body,
        out_shape=jax.ShapeDtypeStruct((1, n_pad), jnp.float32),
    )(deg2)


def _tc_input(x_pad, W0, b0):
    n_pad, d_in = x_pad.shape
    hid = W0.shape[1]
    bn = 1280
    grid = (n_pad // bn,)

    def body(x_ref, w_ref, b_ref, h_ref):
        h = jnp.dot(x_ref[...], w_ref[...],
                    preferred_element_type=jnp.float32) + b_ref[...]
        h_ref[...] = jnp.maximum(h, 0.0)

    return pl.pallas_call(
        body,
        grid=grid,
        in_specs=[pl.BlockSpec((bn, d_in), lambda i: (i, 0)),
                  pl.BlockSpec((d_in, hid), lambda i: (0, 0)),
                  pl.BlockSpec((1, hid), lambda i: (0, 0))],
        out_specs=pl.BlockSpec((bn, hid), lambda i: (i, 0)),
        out_shape=jax.ShapeDtypeStruct((n_pad, hid), jnp.float32),
    )(x_pad, W0, b0.reshape(1, -1))


def _tc_dense(aggA, aggB, h0, cur, W, b):
    n_pad, hid = h0.shape
    bn = 1280
    grid = (n_pad // bn,)

    def body(a_ref, b2_ref, h0_ref, cur_ref, w_ref, bias_ref, o_ref):
        support = ((1.0 - ALPHA) * (a_ref[...] + b2_ref[...])
                   + ALPHA * h0_ref[...])
        out = ((1.0 - BETA) * support
               + BETA * jnp.dot(support, w_ref[...],
                                preferred_element_type=jnp.float32)
               + bias_ref[...])
        o_ref[...] = jnp.maximum(out, 0.0) + cur_ref[...]

    return pl.pallas_call(
        body,
        grid=grid,
        in_specs=[pl.BlockSpec((bn, hid), lambda i: (i, 0)),
                  pl.BlockSpec((bn, hid), lambda i: (i, 0)),
                  pl.BlockSpec((bn, hid), lambda i: (i, 0)),
                  pl.BlockSpec((bn, hid), lambda i: (i, 0)),
                  pl.BlockSpec((hid, hid), lambda i: (0, 0)),
                  pl.BlockSpec((1, hid), lambda i: (0, 0))],
        out_specs=pl.BlockSpec((bn, hid), lambda i: (i, 0)),
        out_shape=jax.ShapeDtypeStruct((n_pad, hid), jnp.float32),
    )(aggA, aggB, h0, cur, W, b.reshape(1, -1))


def _tc_logits(cur, Wp, bp):
    n_pad, hid = cur.shape
    oc = Wp.shape[1]
    bn = 1280
    grid = (n_pad // bn,)

    def body(c_ref, w_ref, b_ref, o_ref):
        logits = jnp.dot(c_ref[...], w_ref[...],
                         preferred_element_type=jnp.float32) + b_ref[...]
        m = jnp.max(logits, axis=1, keepdims=True)
        lse = m + jnp.log(jnp.sum(jnp.exp(logits - m), axis=1,
                                  keepdims=True))
        o_ref[...] = logits - lse

    return pl.pallas_call(
        body,
        grid=grid,
        in_specs=[pl.BlockSpec((bn, hid), lambda i: (i, 0)),
                  pl.BlockSpec((hid, oc), lambda i: (0, 0)),
                  pl.BlockSpec((1, oc), lambda i: (0, 0))],
        out_specs=pl.BlockSpec((bn, oc), lambda i: (i, 0)),
        out_shape=jax.ShapeDtypeStruct((n_pad, oc), jnp.float32),
    )(cur, Wp, bp)


def kernel(x, edge_index, edge_attr, W0, b0, Wc, bc, W_out, b_out):
    N, d_in = x.shape
    hid = W0.shape[1]
    L = Wc.shape[0]
    out_c = W_out.shape[1]
    E = edge_index.shape[1]

    nw = NC * NS
    unit_n = NS * CHD
    n_pad = ((N + unit_n - 1) // unit_n) * unit_n
    e_f = E + N
    unit_e = nw * CH * 8
    nit8 = (e_f + unit_e - 1) // unit_e
    e_pad = nit8 * unit_e
    while e_pad % (nw * CHD) != 0:
        nit8 += 1
        e_pad = nit8 * unit_e
    pt = e_pad // nw
    nit = pt // CH
    pad_e = e_pad - e_f

    row = edge_index[0].astype(jnp.int32)
    col = edge_index[1].astype(jnp.int32)
    loop_idx = jnp.arange(N, dtype=jnp.int32)
    zpad_i = jnp.zeros((pad_e,), jnp.int32)
    row_f = jnp.concatenate([row, loop_idx, zpad_i])
    col_f = jnp.concatenate([col, loop_idx, zpad_i])
    w_f = jnp.concatenate([edge_attr.astype(jnp.float32),
                           jnp.ones((N,), jnp.float32),
                           jnp.zeros((pad_e,), jnp.float32)])
    order = jnp.argsort(row_f)
    row_f = row_f[order]
    col_f = col_f[order]
    w_f = w_f[order]
    col3 = col_f.reshape(nw, e_pad // nw // CHD, CHD)
    w3 = w_f.reshape(nw, e_pad // nw // CHD, CHD)
    row2 = row_f.reshape(nw, pt)
    col2 = col_f.reshape(nw, pt)
    w2 = w_f.reshape(nw, pt)

    deg2 = _make_deg_kernel(e_pad, n_pad)(col3, w3)
    dinv = _tc_dinv(deg2.reshape(NC, n_pad)).reshape(n_pad)
    norm2 = _make_norm_kernel(e_pad, n_pad)(dinv, row2, col2, w2)

    rowr = row_f.reshape(nw * nit, CH)
    colr = col_f.reshape(nw * nit, CH)
    nrmr = norm2.reshape(nw * nit, CH)

    x_pad = jnp.pad(x, ((0, n_pad - N), (0, 0)))
    h0 = _tc_input(x_pad, W0, b0)

    layer_k = _make_layer_kernel(e_pad, n_pad, hid)
    cur = h0
    for l in range(L):
        agg2 = layer_k(cur, rowr, colr, nrmr)
        cur = _tc_dense(agg2[:n_pad], agg2[n_pad:], h0, cur,
                        Wc[l], bc[l])

    pad_c = 128 - out_c
    Wp = jnp.pad(W_out, ((0, 0), (0, pad_c)))
    bp = jnp.pad(b_out, (0, pad_c), constant_values=-1e30)
    ls = _tc_logits(cur, Wp, bp.reshape(1, -1))
    return ls[:N, :out_c]


# speedup vs baseline: 1.7298x; 1.7298x over previous
import functools

import jax
import jax.numpy as jnp
from jax import lax
from jax.experimental import pallas as pl
from jax.experimental.pallas import tpu as pltpu
from jax.experimental.pallas import tpu_sc as plsc

ALPHA = 0.1
BETA = 0.5
NC = 2
NS = 16
LANES = 16
CH = 48
CHD = 128


def _sc_mesh():
    return plsc.VectorSubcoreMesh(
        core_axis_name="c", subcore_axis_name="s",
        num_cores=NC, num_subcores=NS)


_SC_PARAMS = pltpu.CompilerParams(needs_layout_passes=False)


def _make_deg_kernel(e_pad, n_pad):
    nw = NC * NS
    pt = e_pad // nw
    nit = pt // CHD
    slc = n_pad // NS

    @functools.partial(
        pl.kernel,
        out_type=jax.ShapeDtypeStruct((NC * n_pad,), jnp.float32),
        mesh=_sc_mesh(),
        compiler_params=_SC_PARAMS,
        scratch_types=[
            pltpu.VMEM((nit, CHD), jnp.int32),
            pltpu.VMEM((nit, CHD), jnp.float32),
            pltpu.VMEM((slc,), jnp.float32),
            pltpu.VMEM_SHARED((n_pad,), jnp.float32),
        ],
    )
    def deg_kernel(col_hbm, w_hbm, out_hbm, col2_v, w2_v, zb, s_deg):
        c = lax.axis_index("c")
        s = lax.axis_index("s")
        wid = c * NS + s

        def zero_body(i, carry):
            zb[pl.ds(i * LANES, LANES)] = jnp.zeros((LANES,), jnp.float32)
            return carry
        lax.fori_loop(0, slc // LANES, zero_body, 0)
        pltpu.sync_copy(zb, s_deg.at[pl.ds(s * slc, slc)])
        pltpu.sync_copy(col_hbm.at[wid], col2_v)
        pltpu.sync_copy(w_hbm.at[wid], w2_v)
        plsc.subcore_barrier()

        def edge_body(it, carry):
            pltpu.sync_copy(w2_v.at[it], s_deg.at[col2_v.at[it]], add=True)
            return carry
        lax.fori_loop(0, nit, edge_body, 0)
        plsc.subcore_barrier()
        pltpu.sync_copy(s_deg.at[pl.ds(s * slc, slc)],
                        out_hbm.at[pl.ds(c * n_pad + s * slc, slc)])

    return deg_kernel


def _make_norm_kernel(e_pad, n_pad):
    nw = NC * NS
    pt = e_pad // nw

    @functools.partial(
        pl.kernel,
        out_type=jax.ShapeDtypeStruct((nw, pt), jnp.float32),
        mesh=_sc_mesh(),
        compiler_params=_SC_PARAMS,
        scratch_types=[
            pltpu.VMEM((n_pad,), jnp.float32),
            pltpu.VMEM((pt,), jnp.int32),
            pltpu.VMEM((pt,), jnp.int32),
            pltpu.VMEM((pt,), jnp.float32),
            pltpu.VMEM((pt,), jnp.float32),
        ],
    )
    def norm_kernel(dinv_hbm, row_hbm, col_hbm, w_hbm, out_hbm,
                    dinv_v, row_v, col_v, w_v, nrm_v):
        c = lax.axis_index("c")
        s = lax.axis_index("s")
        wid = c * NS + s
        pltpu.sync_copy(dinv_hbm, dinv_v)
        pltpu.sync_copy(row_hbm.at[wid], row_v)
        pltpu.sync_copy(col_hbm.at[wid], col_v)
        pltpu.sync_copy(w_hbm.at[wid], w_v)

        def grp(g, carry):
            sl = pl.ds(g * LANES, LANES)
            dr = plsc.load_gather(dinv_v, [row_v[sl]])
            dc = plsc.load_gather(dinv_v, [col_v[sl]])
            nrm_v[sl] = dr * w_v[sl] * dc
            return carry
        lax.fori_loop(0, pt // LANES, grp, 0)
        pltpu.sync_copy(nrm_v, out_hbm.at[wid])

    return norm_kernel


def _make_layer_kernel(e_pad, n_pad, hid):
    nw = NC * NS
    pt = e_pad // nw
    nit = pt // CH
    assert nit % 8 == 0
    slc = n_pad // NS
    nz = slc // CH
    nz_rem = slc % CH
    kreg = hid // LANES

    @functools.partial(
        pl.kernel,
        out_type=jax.ShapeDtypeStruct((NC * n_pad, hid), jnp.float32),
        mesh=_sc_mesh(),
        compiler_params=_SC_PARAMS,
        scratch_types=(
            [pltpu.VMEM((CH,), jnp.int32) for _ in range(8)]
            + [pltpu.VMEM((CH,), jnp.int32) for _ in range(8)]
            + [pltpu.VMEM((CH,), jnp.float32) for _ in range(8)]
            + [pltpu.VMEM((CH, hid), jnp.float32) for _ in range(4)]
            + [pltpu.VMEM((CH, hid), jnp.float32) for _ in range(2)]
            + [pltpu.VMEM_SHARED((n_pad, hid), jnp.float32)]
            + [pltpu.SemaphoreType.DMA for _ in range(8)]
            + [pltpu.SemaphoreType.DMA for _ in range(4)]
            + [pltpu.SemaphoreType.DMA for _ in range(2)]
        ),
    )
    def layer_kernel(cur_hbm, row_hbm, col_hbm, nrm_hbm, out_hbm, *refs):
        rb = refs[0:8]
        cb = refs[8:16]
        nb_ = refs[16:24]
        gb = refs[24:28]
        sb = refs[28:30]
        s_agg = refs[30]
        esem = refs[31:39]
        gsem = refs[39:43]
        ssem = refs[43:45]
        c = lax.axis_index("c")
        s = lax.axis_index("s")
        wid = c * NS + s

        def fetch(t, q):
            return (pltpu.async_copy(row_hbm.at[wid * nit + t], rb[q],
                                     esem[q]),
                    pltpu.async_copy(col_hbm.at[wid * nit + t], cb[q],
                                     esem[q]),
                    pltpu.async_copy(nrm_hbm.at[wid * nit + t], nb_[q],
                                     esem[q]))

        for buf in sb:
            def zrow(i, carry, buf=buf):
                for k in range(kreg):
                    buf[i, pl.ds(k * LANES, LANES)] = jnp.zeros(
                        (LANES,), jnp.float32)
                return carry
            lax.fori_loop(0, CH, zrow, 0)
        for j in range(nz):
            pltpu.sync_copy(sb[0], s_agg.at[pl.ds(s * slc + j * CH, CH)])
        if nz_rem:
            pltpu.sync_copy(
                sb[0].at[pl.ds(0, nz_rem)],
                s_agg.at[pl.ds(s * slc + nz * CH, nz_rem)])
        plsc.subcore_barrier()

        for q in range(6):
            fetch(q, q)
        ewait = [(pltpu.make_async_copy(row_hbm.at[0], rb[q], esem[q]),
                  pltpu.make_async_copy(col_hbm.at[0], cb[q], esem[q]),
                  pltpu.make_async_copy(nrm_hbm.at[0], nb_[q], esem[q]))
                 for q in range(8)]
        for q in range(4):
            for p in ewait[q]:
                p.wait()
        cps = [pltpu.async_copy(cur_hbm.at[rb[q]], gb[q], gsem[q])
               for q in range(4)]
        dps = [pltpu.async_copy(sb[q], s_agg.at[cb[q]], ssem[q], add=True)
               for q in range(2)]

        def phase(j, carry):
            for b8 in range(8):
                it = 8 * j + b8
                q4 = b8 & 3
                q2 = b8 & 1
                cps[q4].wait()
                dps[q2].wait()

                @plsc.parallel_loop(0, CH, step=4, unroll=4)
                def scale(e0):
                    for du in range(4):
                        e = e0 + du
                        nrm = plsc.load_gather(
                            nb_[b8], [jnp.full((LANES,), e, jnp.int32)])
                        for k in range(kreg):
                            sl = pl.ds(k * LANES, LANES)
                            sb[q2][e, sl] = gb[q4][e, sl] * nrm

                pltpu.async_copy(sb[q2], s_agg.at[cb[b8]], ssem[q2],
                                 add=True)

                @pl.when(it + 4 < nit)
                def _():
                    for p in ewait[(b8 + 4) & 7]:
                        p.wait()
                    pltpu.async_copy(cur_hbm.at[rb[(b8 + 4) & 7]],
                                     gb[q4], gsem[q4])

                @pl.when(it + 6 < nit)
                def _():
                    fetch(it + 6, (b8 + 6) & 7)
            return carry
        lax.fori_loop(0, nit // 8, phase, 0)
        dps[0].wait()
        dps[1].wait()
        plsc.subcore_barrier()
        pltpu.sync_copy(s_agg.at[pl.ds(s * slc, slc)],
                        out_hbm.at[pl.ds(c * n_pad + s * slc, slc)])

    return layer_kernel


def _tc_dinv(deg2):
    n_pad = deg2.shape[1]

    def body(deg_ref, dinv_ref):
        deg = deg_ref[0:1, :] + deg_ref[1:2, :]
        dinv_ref[...] = jnp.where(
            deg > 0, lax.rsqrt(jnp.maximum(deg, 1e-12)), 0.0)

    return pl.pallas_call(
        body,
        out_shape=jax.ShapeDtypeStruct((1, n_pad), jnp.float32),
    )(deg2)


def _tc_input(x_pad, W0, b0):
    n_pad, d_in = x_pad.shape
    hid = W0.shape[1]
    bn = 1280
    grid = (n_pad // bn,)

    def body(x_ref, w_ref, b_ref, h_ref):
        h = jnp.dot(x_ref[...], w_ref[...],
                    preferred_element_type=jnp.float32) + b_ref[...]
        h_ref[...] = jnp.maximum(h, 0.0)

    return pl.pallas_call(
        body,
        grid=grid,
        in_specs=[pl.BlockSpec((bn, d_in), lambda i: (i, 0)),
                  pl.BlockSpec((d_in, hid), lambda i: (0, 0)),
                  pl.BlockSpec((1, hid), lambda i: (0, 0))],
        out_specs=pl.BlockSpec((bn, hid), lambda i: (i, 0)),
        out_shape=jax.ShapeDtypeStruct((n_pad, hid), jnp.float32),
    )(x_pad, W0, b0.reshape(1, -1))


def _tc_dense(aggA, aggB, h0, cur, W, b):
    n_pad, hid = h0.shape
    bn = 1280
    grid = (n_pad // bn,)

    def body(a_ref, b2_ref, h0_ref, cur_ref, w_ref, bias_ref, o_ref):
        support = ((1.0 - ALPHA) * (a_ref[...] + b2_ref[...])
                   + ALPHA * h0_ref[...])
        out = ((1.0 - BETA) * support
               + BETA * jnp.dot(support, w_ref[...],
                                preferred_element_type=jnp.float32)
               + bias_ref[...])
        o_ref[...] = jnp.maximum(out, 0.0) + cur_ref[...]

    return pl.pallas_call(
        body,
        grid=grid,
        in_specs=[pl.BlockSpec((bn, hid), lambda i: (i, 0)),
                  pl.BlockSpec((bn, hid), lambda i: (i, 0)),
                  pl.BlockSpec((bn, hid), lambda i: (i, 0)),
                  pl.BlockSpec((bn, hid), lambda i: (i, 0)),
                  pl.BlockSpec((hid, hid), lambda i: (0, 0)),
                  pl.BlockSpec((1, hid), lambda i: (0, 0))],
        out_specs=pl.BlockSpec((bn, hid), lambda i: (i, 0)),
        out_shape=jax.ShapeDtypeStruct((n_pad, hid), jnp.float32),
    )(aggA, aggB, h0, cur, W, b.reshape(1, -1))


def _tc_logits(cur, Wp, bp):
    n_pad, hid = cur.shape
    oc = Wp.shape[1]
    bn = 1280
    grid = (n_pad // bn,)

    def body(c_ref, w_ref, b_ref, o_ref):
        logits = jnp.dot(c_ref[...], w_ref[...],
                         preferred_element_type=jnp.float32) + b_ref[...]
        m = jnp.max(logits, axis=1, keepdims=True)
        lse = m + jnp.log(jnp.sum(jnp.exp(logits - m), axis=1,
                                  keepdims=True))
        o_ref[...] = logits - lse

    return pl.pallas_call(
        body,
        grid=grid,
        in_specs=[pl.BlockSpec((bn, hid), lambda i: (i, 0)),
                  pl.BlockSpec((hid, oc), lambda i: (0, 0)),
                  pl.BlockSpec((1, oc), lambda i: (0, 0))],
        out_specs=pl.BlockSpec((bn, oc), lambda i: (i, 0)),
        out_shape=jax.ShapeDtypeStruct((n_pad, oc), jnp.float32),
    )(cur, Wp, bp)


def kernel(x, edge_index, edge_attr, W0, b0, Wc, bc, W_out, b_out):
    N, d_in = x.shape
    hid = W0.shape[1]
    L = Wc.shape[0]
    out_c = W_out.shape[1]
    E = edge_index.shape[1]

    nw = NC * NS
    unit_n = NS * CHD
    n_pad = ((N + unit_n - 1) // unit_n) * unit_n
    e_f = E + N
    unit_e = nw * CH * 8
    nit8 = (e_f + unit_e - 1) // unit_e
    e_pad = nit8 * unit_e
    while e_pad % (nw * CHD) != 0:
        nit8 += 1
        e_pad = nit8 * unit_e
    pt = e_pad // nw
    nit = pt // CH
    pad_e = e_pad - e_f

    row = edge_index[0].astype(jnp.int32)
    col = edge_index[1].astype(jnp.int32)
    loop_idx = jnp.arange(N, dtype=jnp.int32)
    zpad_i = jnp.zeros((pad_e,), jnp.int32)
    row_f = jnp.concatenate([row, loop_idx, zpad_i])
    col_f = jnp.concatenate([col, loop_idx, zpad_i])
    w_f = jnp.concatenate([edge_attr.astype(jnp.float32),
                           jnp.ones((N,), jnp.float32),
                           jnp.zeros((pad_e,), jnp.float32)])
    col3 = col_f.reshape(nw, e_pad // nw // CHD, CHD)
    w3 = w_f.reshape(nw, e_pad // nw // CHD, CHD)
    row2 = row_f.reshape(nw, pt)
    col2 = col_f.reshape(nw, pt)
    w2 = w_f.reshape(nw, pt)

    deg2 = _make_deg_kernel(e_pad, n_pad)(col3, w3)
    dinv = _tc_dinv(deg2.reshape(NC, n_pad)).reshape(n_pad)
    norm2 = _make_norm_kernel(e_pad, n_pad)(dinv, row2, col2, w2)

    rowr = row_f.reshape(nw * nit, CH)
    colr = col_f.reshape(nw * nit, CH)
    nrmr = norm2.reshape(nw * nit, CH)

    x_pad = jnp.pad(x, ((0, n_pad - N), (0, 0)))
    h0 = _tc_input(x_pad, W0, b0)

    layer_k = _make_layer_kernel(e_pad, n_pad, hid)
    cur = h0
    for l in range(L):
        agg2 = layer_k(cur, rowr, colr, nrmr)
        cur = _tc_dense(agg2[:n_pad], agg2[n_pad:], h0, cur,
                        Wc[l], bc[l])

    pad_c = 128 - out_c
    Wp = jnp.pad(W_out, ((0, 0), (0, pad_c)))
    bp = jnp.pad(b_out, (0, pad_c), constant_values=-1e30)
    ls = _tc_logits(cur, Wp, bp.reshape(1, -1))
    return ls[:N, :out_c]
